# baseline (speedup 1.0000x reference)
import jax
import jax.numpy as jnp
from jax.experimental import pallas as pl
from jax.experimental.pallas import tpu as pltpu


def _body(x_ref, tf_ref, w1t_ref, b1_ref, e_ref, w1k_ref, w2k_ref, b2r_ref,
          o_ref, c2_ref):
    j = pl.program_id(1)

    @pl.when(j == 0)
    def _():
        ct = jax.lax.dot_general(w1t_ref[...], tf_ref[0],
                                 (((0,), (1,)), ((), ())),
                                 preferred_element_type=jnp.float32)
        ct = ct + b1_ref[...]
        c2_ref[...] = jax.lax.dot_general(e_ref[...], ct,
                                          (((1,), (0,)), ((), ())),
                                          preferred_element_type=jnp.float32)

    x = x_ref[0]
    x01 = jnp.tanh(x[0:2])
    xs = x01.reshape(16, x01.shape[-1]).astype(jnp.bfloat16)
    z = x[2]

    h = jax.lax.dot_general(w1k_ref[...], xs,
                            (((1,), (0,)), ((), ())),
                            preferred_element_type=jnp.float32)
    c2b = c2_ref[...].astype(jnp.bfloat16)
    h = jnp.maximum(h.astype(jnp.bfloat16) + c2b, jnp.bfloat16(0.0))
    raw = jax.lax.dot_general(w2k_ref[...], h,
                              (((1,), (0,)), ((), ())),
                              preferred_element_type=jnp.float32)
    raw = raw + b2r_ref[...]
    sp = jnp.maximum(raw, 0.0) + jnp.log1p(jnp.exp(-jnp.abs(raw))) + 1e-3

    dxl2, dxl1, dxr1, dxr2 = sp[0:8], sp[8:16], sp[16:24], sp[24:32]
    dyl2, dyl1, dyr1, dyr2 = sp[32:40], sp[40:48], sp[48:56], sp[56:64]
    kl = sp[64:72] * 2.0
    kr = sp[72:80] * 2.0
    xL1 = -dxl1
    xL2 = xL1 - dxl2
    xL3 = xL2 - 10000.0
    xR1 = dxr1
    xR2 = dxr1 + dxr2
    xR3 = xR2 + 10000.0
    yL1 = -dyl1
    yL2 = yL1 - dyl2
    yL3 = yL2 - kl * 10000.0
    yR1 = dyr1
    yR2 = dyr1 + dyr2
    yR3 = yR2 + kr * 10000.0

    zc = jnp.clip(z, xL3 * 0.99, xR3 * 0.99)
    xls, xrs, yls, yrs = xL3, xL2, yL3, yL2
    for xk, xn, yk, yn in ((xL2, xL1, yL2, yL1),
                           (xL1, xR1, yL1, yR1),
                           (xR1, xR2, yR1, yR2),
                           (xR2, xR3, yR2, yR3)):
        c = zc >= xk
        xls = jnp.where(c, xk, xls)
        xrs = jnp.where(c, xn, xrs)
        yls = jnp.where(c, yk, yls)
        yrs = jnp.where(c, yn, yrs)
    dydx = (yrs - yls) / (xrs - xls)
    o_ref[0] = dydx * (zc - xls) + yls


def kernel(input, t_feat, W1, b1, W2, b2):
    n, P, S, _ = input.shape
    tot = P * S
    L = tot // 8
    Lb = 4096
    xt = input.reshape(n, tot, 3).transpose(0, 2, 1).reshape(n, 3, 8, L)

    eye8 = jnp.eye(8, dtype=jnp.float32)
    w1k = jnp.kron(W1[:2].T, eye8).astype(jnp.bfloat16)
    w2k = jnp.kron(W2.T, eye8).astype(jnp.bfloat16)
    e = jnp.repeat(jnp.eye(64, dtype=jnp.float32), 8, axis=0)
    b2r = jnp.repeat(b2, 8)[:, None]

    grid = (n, L // Lb)
    z_out = pl.pallas_call(
        _body,
        grid=grid,
        in_specs=[
            pl.BlockSpec((1, 3, 8, Lb), lambda i, j: (i, 0, 0, j)),
            pl.BlockSpec((1, 1, t_feat.shape[1]), lambda i, j: (i, 0, 0)),
            pl.BlockSpec((192, 64), lambda i, j: (0, 0)),
            pl.BlockSpec((64, 1), lambda i, j: (0, 0)),
            pl.BlockSpec((512, 64), lambda i, j: (0, 0)),
            pl.BlockSpec((512, 16), lambda i, j: (0, 0)),
            pl.BlockSpec((80, 512), lambda i, j: (0, 0)),
            pl.BlockSpec((80, 1), lambda i, j: (0, 0)),
        ],
        out_specs=pl.BlockSpec((1, 8, Lb), lambda i, j: (i, 0, j)),
        out_shape=jax.ShapeDtypeStruct((n, 8, L), jnp.float32),
        scratch_shapes=[pltpu.VMEM((512, 1), jnp.float32)],
    )(xt, t_feat[:, None, :], W1[2:], b1[:, None], e, w1k, w2k, b2r)
    z4 = z_out.reshape(n, P, S, 1)
    return jnp.concatenate([input[..., :2], z4], axis=-1)


# speedup vs baseline: 1.1990x; 1.1990x over previous
import jax
import jax.numpy as jnp
from jax.experimental import pallas as pl
from jax.experimental.pallas import tpu as pltpu


def _body(x_ref, tf_ref, w1t_ref, b1_ref, e_ref, w1k_ref, w2k_ref, b2r_ref,
          o_ref, c2_ref):
    j = pl.program_id(1)

    @pl.when(j == 0)
    def _():
        ct = jax.lax.dot_general(w1t_ref[...], tf_ref[0],
                                 (((0,), (1,)), ((), ())),
                                 preferred_element_type=jnp.float32)
        ct = ct + b1_ref[...]
        c2_ref[...] = jax.lax.dot_general(e_ref[...], ct,
                                          (((1,), (0,)), ((), ())),
                                          preferred_element_type=jnp.float32)

    x = x_ref[0]
    x01 = jnp.tanh(x[0:2])
    xs = x01.reshape(16, x01.shape[-1]).astype(jnp.bfloat16)
    z = x[2]

    h = jax.lax.dot_general(w1k_ref[...], xs,
                            (((1,), (0,)), ((), ())),
                            preferred_element_type=jnp.float32)
    h = jnp.maximum(h + c2_ref[...], 0.0).astype(jnp.bfloat16)
    raw = jax.lax.dot_general(w2k_ref[...], h,
                              (((1,), (0,)), ((), ())),
                              preferred_element_type=jnp.float32)
    raw = raw + b2r_ref[...]
    sp = jnp.maximum(raw, 0.0) + jnp.log1p(jnp.exp(-jnp.abs(raw))) + 1e-3

    dxl2, dxl1, dxr1, dxr2 = sp[0:8], sp[8:16], sp[16:24], sp[24:32]
    dyl2, dyl1, dyr1, dyr2 = sp[32:40], sp[40:48], sp[48:56], sp[56:64]
    kl = sp[64:72] * 2.0
    kr = sp[72:80] * 2.0
    xL1 = -dxl1
    xL2 = xL1 - dxl2
    xL3 = xL2 - 10000.0
    xR1 = dxr1
    xR2 = dxr1 + dxr2
    xR3 = xR2 + 10000.0
    yL1 = -dyl1
    yL2 = yL1 - dyl2
    yL3 = yL2 - kl * 10000.0
    yR1 = dyr1
    yR2 = dyr1 + dyr2
    yR3 = yR2 + kr * 10000.0

    zc = jnp.clip(z, xL3 * 0.99, xR3 * 0.99)
    xls, xrs, yls, yrs = xL3, xL2, yL3, yL2
    for xk, xn, yk, yn in ((xL2, xL1, yL2, yL1),
                           (xL1, xR1, yL1, yR1),
                           (xR1, xR2, yR1, yR2),
                           (xR2, xR3, yR2, yR3)):
        c = zc >= xk
        xls = jnp.where(c, xk, xls)
        xrs = jnp.where(c, xn, xrs)
        yls = jnp.where(c, yk, yls)
        yrs = jnp.where(c, yn, yrs)
    dydx = (yrs - yls) / (xrs - xls)
    o_ref[0] = dydx * (zc - xls) + yls


def kernel(input, t_feat, W1, b1, W2, b2):
    n, P, S, _ = input.shape
    tot = P * S
    L = tot // 8
    Lb = 2048
    xt = input.transpose(0, 3, 1, 2).reshape(n, 3, 8, L)

    eye8 = jnp.eye(8, dtype=jnp.float32)
    w1k = jnp.kron(W1[:2].T, eye8).astype(jnp.bfloat16)
    w2k = jnp.kron(W2.T, eye8).astype(jnp.bfloat16)
    e = jnp.repeat(jnp.eye(64, dtype=jnp.float32), 8, axis=0)
    b2r = jnp.repeat(b2, 8)[:, None]

    grid = (n, L // Lb)
    z_out = pl.pallas_call(
        _body,
        grid=grid,
        in_specs=[
            pl.BlockSpec((1, 3, 8, Lb), lambda i, j: (i, 0, 0, j)),
            pl.BlockSpec((1, 1, t_feat.shape[1]), lambda i, j: (i, 0, 0)),
            pl.BlockSpec((192, 64), lambda i, j: (0, 0)),
            pl.BlockSpec((64, 1), lambda i, j: (0, 0)),
            pl.BlockSpec((512, 64), lambda i, j: (0, 0)),
            pl.BlockSpec((512, 16), lambda i, j: (0, 0)),
            pl.BlockSpec((80, 512), lambda i, j: (0, 0)),
            pl.BlockSpec((80, 1), lambda i, j: (0, 0)),
        ],
        out_specs=pl.BlockSpec((1, 8, Lb), lambda i, j: (i, 0, j)),
        out_shape=jax.ShapeDtypeStruct((n, 8, L), jnp.float32),
        scratch_shapes=[pltpu.VMEM((512, 1), jnp.float32)],
    )(xt, t_feat[:, None, :], W1[2:], b1[:, None], e, w1k, w2k, b2r)
    z4 = z_out.reshape(n, P, S, 1)
    return jnp.concatenate([input[..., :2], z4], axis=-1)


# speedup vs baseline: 1.2207x; 1.0181x over previous
import jax
import jax.numpy as jnp
from jax.experimental import pallas as pl
from jax.experimental.pallas import tpu as pltpu


def _body(x_ref, tf_ref, w1t_ref, b1_ref, e_ref, w1k_ref, w2k_ref, b2r_ref,
          o_ref, c2_ref):
    j = pl.program_id(1)

    @pl.when(j == 0)
    def _():
        ct = jax.lax.dot_general(w1t_ref[...], tf_ref[0],
                                 (((0,), (1,)), ((), ())),
                                 preferred_element_type=jnp.float32)
        ct = ct + b1_ref[...]
        c2_ref[...] = jax.lax.dot_general(e_ref[...], ct,
                                          (((1,), (0,)), ((), ())),
                                          preferred_element_type=jnp.float32)

    x = x_ref[0]
    x01 = jnp.tanh(x[0:2])
    xs = x01.reshape(16, x01.shape[-1]).astype(jnp.bfloat16)
    z = x[2]

    h = jax.lax.dot_general(w1k_ref[...], xs,
                            (((1,), (0,)), ((), ())),
                            preferred_element_type=jnp.float32)
    h = jnp.maximum(h + c2_ref[...], 0.0).astype(jnp.bfloat16)
    raw = jax.lax.dot_general(w2k_ref[...], h,
                              (((1,), (0,)), ((), ())),
                              preferred_element_type=jnp.float32)
    raw = raw + b2r_ref[...]
    sp = jnp.maximum(raw, 0.0) + jnp.log1p(jnp.exp(-jnp.abs(raw))) + 1e-3

    dxl2, dxl1, dxr1, dxr2 = sp[0:8], sp[8:16], sp[16:24], sp[24:32]
    dyl2, dyl1, dyr1, dyr2 = sp[32:40], sp[40:48], sp[48:56], sp[56:64]
    kl = sp[64:72] * 2.0
    kr = sp[72:80] * 2.0
    xL1 = -dxl1
    xL2 = xL1 - dxl2
    xL3 = xL2 - 10000.0
    xR1 = dxr1
    xR2 = dxr1 + dxr2
    xR3 = xR2 + 10000.0
    yL1 = -dyl1
    yL2 = yL1 - dyl2
    yL3 = yL2 - kl * 10000.0
    yR1 = dyr1
    yR2 = dyr1 + dyr2
    yR3 = yR2 + kr * 10000.0

    zc = jnp.clip(z, xL3 * 0.99, xR3 * 0.99)
    xls, xrs, yls, yrs = xL3, xL2, yL3, yL2
    for xk, xn, yk, yn in ((xL2, xL1, yL2, yL1),
                           (xL1, xR1, yL1, yR1),
                           (xR1, xR2, yR1, yR2),
                           (xR2, xR3, yR2, yR3)):
        c = zc >= xk
        xls = jnp.where(c, xk, xls)
        xrs = jnp.where(c, xn, xrs)
        yls = jnp.where(c, yk, yls)
        yrs = jnp.where(c, yn, yrs)
    dydx = (yrs - yls) / (xrs - xls)
    o_ref[0] = dydx * (zc - xls) + yls


def kernel(input, t_feat, W1, b1, W2, b2):
    n, P, S, _ = input.shape
    tot = P * S
    L = tot // 8
    Lb = 2048
    xt = input.transpose(0, 3, 1, 2).reshape(n, 3, 8, L)

    eye8 = jnp.eye(8, dtype=jnp.float32)
    w1k = jnp.kron(W1[:2].T, eye8).astype(jnp.bfloat16)
    w2k = jnp.kron(W2.T, eye8).astype(jnp.bfloat16)
    e = jnp.repeat(jnp.eye(64, dtype=jnp.float32), 8, axis=0)
    b2r = jnp.repeat(b2, 8)[:, None]

    grid = (n, L // Lb)
    z_out = pl.pallas_call(
        _body,
        grid=grid,
        in_specs=[
            pl.BlockSpec((1, 3, 8, Lb), lambda i, j: (i, 0, 0, j)),
            pl.BlockSpec((1, 1, t_feat.shape[1]), lambda i, j: (i, 0, 0)),
            pl.BlockSpec((192, 64), lambda i, j: (0, 0)),
            pl.BlockSpec((64, 1), lambda i, j: (0, 0)),
            pl.BlockSpec((512, 64), lambda i, j: (0, 0)),
            pl.BlockSpec((512, 16), lambda i, j: (0, 0)),
            pl.BlockSpec((80, 512), lambda i, j: (0, 0)),
            pl.BlockSpec((80, 1), lambda i, j: (0, 0)),
        ],
        out_specs=pl.BlockSpec((1, 8, Lb), lambda i, j: (i, 0, j)),
        out_shape=jax.ShapeDtypeStruct((n, 8, L), jnp.float32),
        scratch_shapes=[pltpu.VMEM((512, 1), jnp.float32)],
    )(xt, t_feat[:, None, :], W1[2:], b1[:, None], e, w1k, w2k, b2r)
    return input.at[..., 2].set(z_out.reshape(n, P, S))


# speedup vs baseline: 1.2503x; 1.0243x over previous
import jax
import jax.numpy as jnp
from jax.experimental import pallas as pl
from jax.experimental.pallas import tpu as pltpu


def _body(x_ref, tf_ref, w1t_ref, b1_ref, e_ref, w1k_ref, w2k_ref, b2r_ref,
          o_ref, c2_ref):
    j = pl.program_id(1)

    @pl.when(j == 0)
    def _():
        ct = jax.lax.dot_general(w1t_ref[...], tf_ref[0],
                                 (((0,), (1,)), ((), ())),
                                 preferred_element_type=jnp.float32)
        ct = ct + b1_ref[...]
        c2_ref[...] = jax.lax.dot_general(e_ref[...], ct,
                                          (((1,), (0,)), ((), ())),
                                          preferred_element_type=jnp.float32)

    x = x_ref[0]
    x01 = jnp.tanh(x[0:2])
    xs = x01.reshape(16, x01.shape[-1]).astype(jnp.bfloat16)
    z = x[2]

    h = jax.lax.dot_general(w1k_ref[...], xs,
                            (((1,), (0,)), ((), ())),
                            preferred_element_type=jnp.float32)
    h = jnp.maximum(h + c2_ref[...], 0.0).astype(jnp.bfloat16)
    raw = jax.lax.dot_general(w2k_ref[...], h,
                              (((1,), (0,)), ((), ())),
                              preferred_element_type=jnp.float32)
    raw = raw + b2r_ref[...]
    sp = jnp.maximum(raw, 0.0) + jnp.log1p(jnp.exp(-jnp.abs(raw))) + 1e-3

    dxl2, dxl1, dxr1, dxr2 = sp[0:8], sp[8:16], sp[16:24], sp[24:32]
    dyl2, dyl1, dyr1, dyr2 = sp[32:40], sp[40:48], sp[48:56], sp[56:64]
    kl = sp[64:72] * 2.0
    kr = sp[72:80] * 2.0
    xL1 = -dxl1
    xL2 = xL1 - dxl2
    xL3 = xL2 - 10000.0
    xR1 = dxr1
    xR2 = dxr1 + dxr2
    xR3 = xR2 + 10000.0
    yL1 = -dyl1
    yL2 = yL1 - dyl2
    yL3 = yL2 - kl * 10000.0
    yR1 = dyr1
    yR2 = dyr1 + dyr2
    yR3 = yR2 + kr * 10000.0

    zc = jnp.clip(z, xL3 * 0.99, xR3 * 0.99)
    xls, xrs, yls, yrs = xL3, xL2, yL3, yL2
    for xk, xn, yk, yn in ((xL2, xL1, yL2, yL1),
                           (xL1, xR1, yL1, yR1),
                           (xR1, xR2, yR1, yR2),
                           (xR2, xR3, yR2, yR3)):
        c = zc >= xk
        xls = jnp.where(c, xk, xls)
        xrs = jnp.where(c, xn, xrs)
        yls = jnp.where(c, yk, yls)
        yrs = jnp.where(c, yn, yrs)
    dydx = (yrs - yls) / (xrs - xls)
    o_ref[0] = dydx * (zc - xls) + yls


def kernel(input, t_feat, W1, b1, W2, b2):
    n, P, S, _ = input.shape
    tot = P * S
    L = tot // 8
    Lb = 8192
    xt = input.transpose(0, 3, 1, 2).reshape(n, 3, 8, L)

    eye8 = jnp.eye(8, dtype=jnp.float32)
    w1k = jnp.kron(W1[:2].T, eye8).astype(jnp.bfloat16)
    w2k = jnp.kron(W2.T, eye8).astype(jnp.bfloat16)
    e = jnp.repeat(jnp.eye(64, dtype=jnp.float32), 8, axis=0)
    b2r = jnp.repeat(b2, 8)[:, None]

    grid = (n, L // Lb)
    z_out = pl.pallas_call(
        _body,
        grid=grid,
        in_specs=[
            pl.BlockSpec((1, 3, 8, Lb), lambda i, j: (i, 0, 0, j)),
            pl.BlockSpec((1, 1, t_feat.shape[1]), lambda i, j: (i, 0, 0)),
            pl.BlockSpec((192, 64), lambda i, j: (0, 0)),
            pl.BlockSpec((64, 1), lambda i, j: (0, 0)),
            pl.BlockSpec((512, 64), lambda i, j: (0, 0)),
            pl.BlockSpec((512, 16), lambda i, j: (0, 0)),
            pl.BlockSpec((80, 512), lambda i, j: (0, 0)),
            pl.BlockSpec((80, 1), lambda i, j: (0, 0)),
        ],
        out_specs=pl.BlockSpec((1, 8, Lb), lambda i, j: (i, 0, j)),
        out_shape=jax.ShapeDtypeStruct((n, 8, L), jnp.float32),
        scratch_shapes=[pltpu.VMEM((512, 1), jnp.float32)],
    )(xt, t_feat[:, None, :], W1[2:], b1[:, None], e, w1k, w2k, b2r)
    return input.at[..., 2].set(z_out.reshape(n, P, S))
